# Initial kernel scaffold; baseline (speedup 1.0000x reference)
import jax
import jax.numpy as jnp
from jax.experimental import pallas as pl


def kernel(input, edge_index, A, B, gamma, beta):
    raise NotImplementedError("write your pallas kernel here")


# speedup vs baseline: 5.5494x; 5.5494x over previous
import functools

import jax
import jax.numpy as jnp
from jax import lax
from jax.experimental import pallas as pl
from jax.experimental.pallas import tpu as pltpu
from jax.experimental.pallas import tpu_sc as plsc

N_NODES = 10000
FDIM = 128
NCORES = 2
NSUB = 16
NTILES = NCORES * NSUB
CHUNK = 80


def _quat_mul_mat(k):
    r, i, j, q = jnp.split(k, 4, axis=1)
    r2 = jnp.concatenate([r, -i, -j, -q], axis=0)
    i2 = jnp.concatenate([i, r, -q, j], axis=0)
    j2 = jnp.concatenate([j, q, r, -i], axis=0)
    k2 = jnp.concatenate([q, -j, i, r], axis=0)
    return jnp.concatenate([r2, i2, j2, k2], axis=1)


def _sc_aggregate(x, src, dst):
    n_edges = src.shape[0]
    per_tile = n_edges // NTILES
    n_chunks = per_tile // CHUNK
    rows_per_tile = 624
    tail_start = NSUB * rows_per_tile
    tail_rows = N_NODES - tail_start
    nz_full = rows_per_tile // CHUNK
    nz_rem = rows_per_tile % CHUNK

    mesh = plsc.VectorSubcoreMesh(core_axis_name="c", subcore_axis_name="s")

    @functools.partial(
        pl.kernel,
        out_type=jax.ShapeDtypeStruct((NCORES, N_NODES, FDIM), jnp.float32),
        mesh=mesh,
        scratch_types=[
            pltpu.VMEM((CHUNK,), jnp.int32),
            pltpu.VMEM((CHUNK,), jnp.int32),
            pltpu.VMEM((CHUNK, FDIM), jnp.float32),
            pltpu.VMEM_SHARED((N_NODES, FDIM), jnp.float32),
            pltpu.SemaphoreType.DMA,
        ],
    )
    def body(x_hbm, src_hbm, dst_hbm, out_hbm, src_v, dst_v, rows_v, acc, sem):
        c = lax.axis_index("c")
        s = lax.axis_index("s")
        wid = c * NSUB + s

        zero16 = jnp.zeros((16,), jnp.float32)

        def zero_row(i, carry):
            for j in range(FDIM // 16):
                rows_v[i, pl.ds(j * 16, 16)] = zero16
            return carry

        lax.fori_loop(0, CHUNK, zero_row, 0)

        row0 = s * rows_per_tile

        def zero_acc(i, carry):
            pltpu.sync_copy(rows_v, acc.at[pl.ds(row0 + i * CHUNK, CHUNK)])
            return carry

        lax.fori_loop(0, nz_full, zero_acc, 0)
        if nz_rem:
            pltpu.sync_copy(
                rows_v.at[pl.ds(0, nz_rem)],
                acc.at[pl.ds(row0 + nz_full * CHUNK, nz_rem)],
            )

        @pl.when(s == NSUB - 1)
        def _():
            pltpu.sync_copy(
                rows_v.at[pl.ds(0, tail_rows)],
                acc.at[pl.ds(tail_start, tail_rows)],
            )

        plsc.subcore_barrier()

        ebase = wid * per_tile

        def step(i, carry):
            b = ebase + i * CHUNK
            pltpu.sync_copy(src_hbm.at[pl.ds(b, CHUNK)], src_v)
            pltpu.sync_copy(dst_hbm.at[pl.ds(b, CHUNK)], dst_v)
            pltpu.async_copy(x_hbm.at[src_v], rows_v, sem).wait()
            pltpu.sync_copy(rows_v, acc.at[dst_v], add=True)
            return carry

        lax.fori_loop(0, n_chunks, step, 0)
        plsc.subcore_barrier()

        pltpu.sync_copy(
            acc.at[pl.ds(row0, rows_per_tile)],
            out_hbm.at[c, pl.ds(row0, rows_per_tile)],
        )

        @pl.when(s == NSUB - 1)
        def _():
            pltpu.sync_copy(
                acc.at[pl.ds(tail_start, tail_rows)],
                out_hbm.at[c, pl.ds(tail_start, tail_rows)],
            )

    return body(x, src, dst)


def _tc_finish(partials, A, B, gamma2d, beta2d):
    def body(p_ref, a_ref, b_ref, g_ref, bt_ref, o_ref):
        agg = p_ref[0] + p_ref[1]
        a_h = _quat_mul_mat(a_ref[...])
        b_h = _quat_mul_mat(b_ref[...])
        zeros = jnp.zeros_like(a_h)
        w = jnp.concatenate(
            [
                jnp.concatenate([a_h, b_h], axis=1),
                jnp.concatenate([zeros, a_h], axis=1),
            ],
            axis=0,
        )
        s = lax.dot_general(
            agg,
            w,
            (((1,), (0,)), ((), ())),
            preferred_element_type=jnp.float32,
            precision=lax.Precision.HIGHEST,
        )
        mean = jnp.mean(s, axis=0, keepdims=True)
        d = s - mean
        var = jnp.mean(d * d, axis=0, keepdims=True)
        o_ref[...] = jnp.tanh(
            d * lax.rsqrt(var + 1e-5) * g_ref[...] + bt_ref[...]
        )

    return pl.pallas_call(
        body,
        out_shape=jax.ShapeDtypeStruct((N_NODES, FDIM), jnp.float32),
    )(partials, A, B, gamma2d, beta2d)


def kernel(input, edge_index, A, B, gamma, beta):
    ei = edge_index.astype(jnp.int32)
    dst = ei[0]
    src = ei[1]
    partials = _sc_aggregate(input, src, dst)
    return _tc_finish(
        partials, A, B, gamma.reshape(1, FDIM), beta.reshape(1, FDIM)
    )
